# baseline (speedup 1.0000x reference)
import functools

import jax
import jax.numpy as jnp
from jax import lax
from jax.experimental import pallas as pl
from jax.experimental.pallas import tpu as pltpu
from jax.experimental.pallas import tpu_sc as plsc

_B, _L, _D = 64, 1024, 128
_SG = 8
_LG = 256
_V = 16
_NU = _LG // _V
_NBUF = 3
_LANES = 16


@jax.jit
def _sc_embed(x, token_table, pos_table):
  mesh = plsc.VectorSubcoreMesh(core_axis_name="c", subcore_axis_name="s")

  @functools.partial(
      pl.kernel,
      mesh=mesh,
      out_type=jax.ShapeDtypeStruct((_B, _L, _D), jnp.float32),
      scratch_types=[
          pltpu.VMEM((_SG, _LG), jnp.int32),
          pltpu.VMEM((_LG, _D), jnp.float32),
          pltpu.VMEM((_SG, _V, _D), jnp.float32),
          pltpu.VMEM((_SG, _V, _D), jnp.float32),
          pltpu.VMEM((_SG, _V, _D), jnp.float32),
          pltpu.SemaphoreType.DMA,
          pltpu.SemaphoreType.DMA,
          pltpu.SemaphoreType.DMA,
          pltpu.SemaphoreType.DMA,
          pltpu.SemaphoreType.DMA,
          pltpu.SemaphoreType.DMA,
          pltpu.SemaphoreType.DMA,
      ],
  )
  def k(x_hbm, tok_hbm, pos_hbm, out_hbm,
        idx_v, pos_v, rows0, rows1, rows2, psem, g0, g1, g2, o0, o1, o2):
    wid = lax.axis_index("s") * 2 + lax.axis_index("c")
    g0row = (wid // 4) * _SG
    l0 = (wid % 4) * _LG

    pltpu.sync_copy(x_hbm.at[pl.ds(g0row, _SG), pl.ds(l0, _LG)], idx_v)
    pos_h = pltpu.async_copy(pos_hbm.at[pl.ds(l0, _LG), :], pos_v, psem)

    rows = [rows0, rows1, rows2]
    gsem = [g0, g1, g2]
    osem = [o0, o1, o2]

    def start(v):
      b = v % _NBUF
      return [
          pltpu.async_copy(
              tok_hbm.at[idx_v.at[s, pl.ds(v * _V, _V)]],
              rows[b].at[s], gsem[b])
          for s in range(_SG)
      ]

    in_flight = {0: start(0), 1: start(1)}
    out_flight = {}
    for v in range(_NU):
      b = v % _NBUF
      if v + 2 < _NU:
        if v - 1 in out_flight:
          for h in out_flight.pop(v - 1):
            h.wait()
        in_flight[v + 2] = start(v + 2)
      for h in in_flight.pop(v):
        h.wait()
      if v == 0:
        pos_h.wait()

      def add_body(i, _):
        for j in range(_D // _LANES):
          o = j * _LANES
          p = pos_v[v * _V + i, pl.ds(o, _LANES)]
          for s in range(_SG):
            rows[b][s, i, pl.ds(o, _LANES)] = (
                rows[b][s, i, pl.ds(o, _LANES)] + p)
        return 0

      lax.fori_loop(0, _V, add_body, 0)

      out_flight[v] = [
          pltpu.async_copy(
              rows[b].at[s],
              out_hbm.at[g0row + s, pl.ds(l0 + v * _V, _V), :], osem[b])
          for s in range(_SG)
      ]

    for v in sorted(out_flight):
      for h in out_flight[v]:
        h.wait()

  return k(x, token_table, pos_table)


def kernel(x, token_table, pos_table):
  return _sc_embed(x, token_table, pos_table)


# speedup vs baseline: 1.0732x; 1.0732x over previous
import functools

import jax
import jax.numpy as jnp
from jax import lax
from jax.experimental import pallas as pl
from jax.experimental.pallas import tpu as pltpu
from jax.experimental.pallas import tpu_sc as plsc

_B, _L, _D = 64, 1024, 128
_SG = 8
_LG = 256
_V = 32
_NU = _LG // _V
_NBUF = 2
_LANES = 16


@jax.jit
def _sc_embed(x, token_table, pos_table):
  mesh = plsc.VectorSubcoreMesh(core_axis_name="c", subcore_axis_name="s")

  @functools.partial(
      pl.kernel,
      mesh=mesh,
      out_type=jax.ShapeDtypeStruct((_B, _L, _D), jnp.float32),
      scratch_types=[
          pltpu.VMEM((_SG, _LG), jnp.int32),
          pltpu.VMEM((_LG, _D), jnp.float32),
          pltpu.VMEM((_SG, _V, _D), jnp.float32),
          pltpu.VMEM((_SG, _V, _D), jnp.float32),
          pltpu.SemaphoreType.DMA,
          pltpu.SemaphoreType.DMA,
          pltpu.SemaphoreType.DMA,
          pltpu.SemaphoreType.DMA,
          pltpu.SemaphoreType.DMA,
      ],
  )
  def k(x_hbm, tok_hbm, pos_hbm, out_hbm,
        idx_v, pos_v, rows0, rows1, psem, g0, g1, o0, o1):
    wid = lax.axis_index("s") * 2 + lax.axis_index("c")
    g0row = (wid // 4) * _SG
    l0 = (wid % 4) * _LG

    pltpu.sync_copy(x_hbm.at[pl.ds(g0row, _SG), pl.ds(l0, _LG)], idx_v)
    pos_h = pltpu.async_copy(pos_hbm.at[pl.ds(l0, _LG), :], pos_v, psem)

    rows = [rows0, rows1]
    gsem = [g0, g1]
    osem = [o0, o1]

    def start(v):
      b = v % _NBUF
      return [
          pltpu.async_copy(
              tok_hbm.at[idx_v.at[s, pl.ds(v * _V, _V)]],
              rows[b].at[s], gsem[b])
          for s in range(_SG)
      ]

    in_flight = {0: start(0)}
    out_flight = {}
    for v in range(_NU):
      b = v % _NBUF
      if v + 1 < _NU:
        if v - 1 in out_flight:
          for h in out_flight.pop(v - 1):
            h.wait()
        in_flight[v + 1] = start(v + 1)
      for h in in_flight.pop(v):
        h.wait()
      if v == 0:
        pos_h.wait()

      def add_body(t, _):
        i = t // (_D // _LANES)
        o = (t % (_D // _LANES)) * _LANES
        p = pos_v[v * _V + i, pl.ds(o, _LANES)]
        for s in range(_SG):
          rows[b][s, i, pl.ds(o, _LANES)] = (
              rows[b][s, i, pl.ds(o, _LANES)] + p)
        return 0

      lax.fori_loop(0, _V * (_D // _LANES), add_body, 0)

      out_flight[v] = [
          pltpu.async_copy(
              rows[b].at[s],
              out_hbm.at[g0row + s, pl.ds(l0 + v * _V, _V), :], osem[b])
          for s in range(_SG)
      ]

    for v in sorted(out_flight):
      for h in out_flight[v]:
        h.wait()

  return k(x, token_table, pos_table)


def kernel(x, token_table, pos_table):
  return _sc_embed(x, token_table, pos_table)


# speedup vs baseline: 1.0940x; 1.0194x over previous
import functools

import jax
import jax.numpy as jnp
from jax import lax
from jax.experimental import pallas as pl
from jax.experimental.pallas import tpu as pltpu
from jax.experimental.pallas import tpu_sc as plsc

_B, _L, _D = 64, 1024, 128
_SG = 8
_LG = 256
_V = 32
_NU = _LG // _V
_NBUF = 3
_LANES = 16


@jax.jit
def _sc_embed(x, token_table, pos_table):
  mesh = plsc.VectorSubcoreMesh(core_axis_name="c", subcore_axis_name="s")

  @functools.partial(
      pl.kernel,
      mesh=mesh,
      out_type=jax.ShapeDtypeStruct((_B, _L, _D), jnp.float32),
      scratch_types=[
          pltpu.VMEM((_SG, _LG), jnp.int32),
          pltpu.VMEM((_SG, _V, _D), jnp.float32),
          pltpu.VMEM((_SG, _V, _D), jnp.float32),
          pltpu.VMEM((_SG, _V, _D), jnp.float32),
          pltpu.VMEM((_V, _D), jnp.float32),
          pltpu.VMEM((_V, _D), jnp.float32),
          pltpu.VMEM((_V, _D), jnp.float32),
          pltpu.SemaphoreType.DMA,
          pltpu.SemaphoreType.DMA,
          pltpu.SemaphoreType.DMA,
          pltpu.SemaphoreType.DMA,
          pltpu.SemaphoreType.DMA,
          pltpu.SemaphoreType.DMA,
          pltpu.SemaphoreType.DMA,
          pltpu.SemaphoreType.DMA,
          pltpu.SemaphoreType.DMA,
      ],
  )
  def k(x_hbm, tok_hbm, pos_hbm, out_hbm, idx_v,
        rows0, rows1, rows2, posb0, posb1, posb2,
        g0, g1, g2, p0, p1, p2, o0, o1, o2):
    wid = lax.axis_index("s") * 2 + lax.axis_index("c")
    g0row = (wid // 4) * _SG
    l0 = (wid % 4) * _LG

    pltpu.sync_copy(x_hbm.at[pl.ds(g0row, _SG), pl.ds(l0, _LG)], idx_v)

    rows = [rows0, rows1, rows2]
    posb = [posb0, posb1, posb2]
    gsem = [g0, g1, g2]
    psem = [p0, p1, p2]
    osem = [o0, o1, o2]

    def start(v):
      b = v % _NBUF
      hs = [
          pltpu.async_copy(
              tok_hbm.at[idx_v.at[s, pl.ds(v * _V, _V)]],
              rows[b].at[s], gsem[b])
          for s in range(_SG)
      ]
      hs.append(pltpu.async_copy(
          pos_hbm.at[pl.ds(l0 + v * _V, _V), :], posb[b], psem[b]))
      return hs

    in_flight = {0: start(0), 1: start(1)}
    out_flight = {}
    for v in range(_NU):
      b = v % _NBUF
      if v + 2 < _NU:
        if v - 1 in out_flight:
          for h in out_flight.pop(v - 1):
            h.wait()
        in_flight[v + 2] = start(v + 2)
      for h in in_flight.pop(v):
        h.wait()

      def add_body(i, _):
        for j in range(_D // _LANES):
          o = j * _LANES
          p = posb[b][i, pl.ds(o, _LANES)]
          for s in range(_SG):
            rows[b][s, i, pl.ds(o, _LANES)] = (
                rows[b][s, i, pl.ds(o, _LANES)] + p)
        return 0

      lax.fori_loop(0, _V, add_body, 0)

      out_flight[v] = [
          pltpu.async_copy(
              rows[b].at[s],
              out_hbm.at[g0row + s, pl.ds(l0 + v * _V, _V), :], osem[b])
          for s in range(_SG)
      ]

    for v in sorted(out_flight):
      for h in out_flight[v]:
        h.wait()

  return k(x, token_table, pos_table)


def kernel(x, token_table, pos_table):
  return _sc_embed(x, token_table, pos_table)
